# baseline (speedup 1.0000x reference)
import jax
import jax.numpy as jnp
from jax.experimental import pallas as pl
from jax.experimental.pallas import tpu as pltpu

_NF = 51
_BLK = 1024
_MOM = 0.1
_EPS = 1e-5


def _fused_kernel(lt_ref, labels_ref, rlp_ref, rcpt_ref, out_ref,
                  pt_scr, ct_scr, rt_scr, w2_scr):
    phase = pl.program_id(0)
    i = pl.program_id(1)

    @pl.when(phase == 0)
    def _pass0():
        @pl.when(i == 0)
        def _init():
            ct_scr[...] = jnp.zeros_like(ct_scr)
            rt_scr[...] = jnp.zeros_like(rt_scr)

        lt = jnp.transpose(lt_ref[...], (1, 0))
        lab = labels_ref[0]
        riota = jax.lax.broadcasted_iota(jnp.int32, (_NF, _BLK), 0)

        ml = jnp.where(riota >= 1, lt, -jnp.inf)
        mx = jnp.max(ml, axis=0, keepdims=True)
        pred = jnp.min(jnp.where(ml == mx, riota, _NF + 1), axis=0,
                       keepdims=True)

        mfull = jnp.maximum(mx, lt[0:1, :])
        e = jnp.exp(lt - mfull)
        probs = e / jnp.sum(e, axis=0, keepdims=True)
        pt_scr[:, pl.ds(i * _BLK, _BLK)] = probs

        fg = lab != 0
        oh_lab = ((riota == lab) & fg).astype(jnp.float32)
        oh_pred = ((riota == pred) & fg).astype(jnp.float32)
        ct_scr[...] += jax.lax.dot_general(
            oh_pred, oh_lab, (((1,), (1,)), ((), ())),
            preferred_element_type=jnp.float32)
        rt_scr[...] += jax.lax.dot_general(
            probs, oh_lab, (((1,), (1,)), ((), ())),
            preferred_element_type=jnp.float32)

    @pl.when(phase == 1)
    def _pass1():
        @pl.when(i == 0)
        def _solve():
            ct = ct_scr[...]
            m_t = ct + rcpt_ref[...]
            cn = m_t / jnp.sum(m_t, axis=1, keepdims=True)
            cnt_col = jnp.sum(ct, axis=0, keepdims=True)
            r2 = jax.lax.broadcasted_iota(jnp.int32, (_NF, _NF), 0)
            c2 = jax.lax.broadcasted_iota(jnp.int32, (_NF, _NF), 1)
            eye = (r2 == c2).astype(jnp.float32)
            sp_col = jnp.sum(rt_scr[...] * eye, axis=0, keepdims=True)
            lp = sp_col / cnt_col
            rlp = rlp_ref[...]
            rlp_new = jnp.where(cnt_col > 0.0,
                                _MOM * lp + (1.0 - _MOM) * rlp, rlp)
            rp = jax.lax.dot_general(
                rlp_new, cn, (((1,), (0,)), ((), ())),
                preferred_element_type=jnp.float32)
            w2_scr[...] = cn / (rp + _EPS)

        pt = pt_scr[:, pl.ds(i * _BLK, _BLK)]
        out_ref[...] = jax.lax.dot_general(
            pt, w2_scr[...], (((0,), (0,)), ((), ())),
            preferred_element_type=jnp.float32)


def kernel(relation_logits, rel_labels, running_labeling_prob,
           running_column_prob, running_label):
    b, nf = relation_logits.shape
    nb = b // _BLK
    labels3 = rel_labels.reshape(nb, 1, _BLK)
    rlp2 = running_labeling_prob.reshape(1, nf)
    rcpt = running_column_prob.T
    return pl.pallas_call(
        _fused_kernel,
        grid=(2, nb),
        in_specs=[
            pl.BlockSpec((_BLK, nf), lambda p, i: (i * (1 - p), 0)),
            pl.BlockSpec((1, 1, _BLK), lambda p, i: (i * (1 - p), 0, 0)),
            pl.BlockSpec((1, nf), lambda p, i: (0, 0)),
            pl.BlockSpec((nf, nf), lambda p, i: (0, 0)),
        ],
        out_specs=pl.BlockSpec((_BLK, nf), lambda p, i: (i * p, 0)),
        out_shape=jax.ShapeDtypeStruct((b, nf), jnp.float32),
        scratch_shapes=[
            pltpu.VMEM((nf, b), jnp.float32),
            pltpu.VMEM((nf, nf), jnp.float32),
            pltpu.VMEM((nf, nf), jnp.float32),
            pltpu.VMEM((nf, nf), jnp.float32),
        ],
    )(relation_logits, labels3, rlp2, rcpt)


# speedup vs baseline: 1.7553x; 1.7553x over previous
import jax
import jax.numpy as jnp
from jax.experimental import pallas as pl
from jax.experimental.pallas import tpu as pltpu

_NF = 51
_BLK = 2048
_MOM = 0.1
_EPS = 1e-5


def _fused_kernel(lt_ref, labels_ref, rlp_ref, rcpt_ref, out_ref,
                  pt_scr, ct_scr, rt_scr, w2_scr):
    phase = pl.program_id(0)
    i = pl.program_id(1)

    @pl.when(phase == 0)
    def _pass0():
        @pl.when(i == 0)
        def _init():
            ct_scr[...] = jnp.zeros_like(ct_scr)
            rt_scr[...] = jnp.zeros_like(rt_scr)

        lt = lt_ref[...]
        lab = labels_ref[0]
        riota = jax.lax.broadcasted_iota(jnp.int32, (_NF, _BLK), 0)

        ml = jnp.where(riota >= 1, lt, -jnp.inf)
        mx = jnp.max(ml, axis=0, keepdims=True)
        pred = jnp.min(jnp.where(ml == mx, riota, _NF + 1), axis=0,
                       keepdims=True)

        mfull = jnp.maximum(mx, lt[0:1, :])
        e = jnp.exp(lt - mfull)
        probs = e / jnp.sum(e, axis=0, keepdims=True)
        pt_scr[:, pl.ds(i * _BLK, _BLK)] = probs

        fg = lab != 0
        oh_lab = ((riota == lab) & fg).astype(jnp.float32)
        oh_pred = ((riota == pred) & fg).astype(jnp.float32)
        ct_scr[...] += jax.lax.dot_general(
            oh_pred, oh_lab, (((1,), (1,)), ((), ())),
            preferred_element_type=jnp.float32)
        rt_scr[...] += jax.lax.dot_general(
            probs, oh_lab, (((1,), (1,)), ((), ())),
            preferred_element_type=jnp.float32)

    @pl.when(phase == 1)
    def _pass1():
        @pl.when(i == 0)
        def _solve():
            ct = ct_scr[...]
            m_t = ct + rcpt_ref[...]
            cn = m_t / jnp.sum(m_t, axis=1, keepdims=True)
            cnt_col = jnp.sum(ct, axis=0, keepdims=True)
            r2 = jax.lax.broadcasted_iota(jnp.int32, (_NF, _NF), 0)
            c2 = jax.lax.broadcasted_iota(jnp.int32, (_NF, _NF), 1)
            eye = (r2 == c2).astype(jnp.float32)
            sp_col = jnp.sum(rt_scr[...] * eye, axis=0, keepdims=True)
            lp = sp_col / cnt_col
            rlp = rlp_ref[...]
            rlp_new = jnp.where(cnt_col > 0.0,
                                _MOM * lp + (1.0 - _MOM) * rlp, rlp)
            rp = jax.lax.dot_general(
                rlp_new, cn, (((1,), (0,)), ((), ())),
                preferred_element_type=jnp.float32)
            w2_scr[...] = cn / (rp + _EPS)

        pt = pt_scr[:, pl.ds(i * _BLK, _BLK)]
        out_ref[...] = jax.lax.dot_general(
            pt, w2_scr[...], (((0,), (0,)), ((), ())),
            preferred_element_type=jnp.float32)


def kernel(relation_logits, rel_labels, running_labeling_prob,
           running_column_prob, running_label):
    b, nf = relation_logits.shape
    nb = b // _BLK
    lt = relation_logits.T
    labels3 = rel_labels.reshape(nb, 1, _BLK)
    rlp2 = running_labeling_prob.reshape(1, nf)
    rcpt = running_column_prob.T
    return pl.pallas_call(
        _fused_kernel,
        grid=(2, nb),
        in_specs=[
            pl.BlockSpec((nf, _BLK), lambda p, i: (0, i * (1 - p))),
            pl.BlockSpec((1, 1, _BLK), lambda p, i: (i * (1 - p), 0, 0)),
            pl.BlockSpec((1, nf), lambda p, i: (0, 0)),
            pl.BlockSpec((nf, nf), lambda p, i: (0, 0)),
        ],
        out_specs=pl.BlockSpec((_BLK, nf), lambda p, i: (i * p, 0)),
        out_shape=jax.ShapeDtypeStruct((b, nf), jnp.float32),
        scratch_shapes=[
            pltpu.VMEM((nf, b), jnp.float32),
            pltpu.VMEM((nf, nf), jnp.float32),
            pltpu.VMEM((nf, nf), jnp.float32),
            pltpu.VMEM((nf, nf), jnp.float32),
        ],
    )(lt, labels3, rlp2, rcpt)


# speedup vs baseline: 2.0676x; 1.1779x over previous
import jax
import jax.numpy as jnp
from jax.experimental import pallas as pl
from jax.experimental.pallas import tpu as pltpu

_NF = 51
_BLK = 4096
_MOM = 0.1
_EPS = 1e-5


def _fused_kernel(lt_ref, labels_ref, rlp_ref, rcpt_ref, out_ref,
                  pt_scr, ct_scr, rt_scr, w2_scr):
    phase = pl.program_id(0)
    i = pl.program_id(1)

    @pl.when(phase == 0)
    def _pass0():
        @pl.when(i == 0)
        def _init():
            ct_scr[...] = jnp.zeros_like(ct_scr)
            rt_scr[...] = jnp.zeros_like(rt_scr)

        lt = lt_ref[...]
        lab = labels_ref[0]
        riota = jax.lax.broadcasted_iota(jnp.int32, (_NF, _BLK), 0)

        ml = jnp.where(riota >= 1, lt, -jnp.inf)
        mx = jnp.max(ml, axis=0, keepdims=True)
        pred = jnp.min(jnp.where(ml == mx, riota, _NF + 1), axis=0,
                       keepdims=True)

        mfull = jnp.maximum(mx, lt[0:1, :])
        e = jnp.exp(lt - mfull)
        probs = e / jnp.sum(e, axis=0, keepdims=True)
        pt_scr[:, pl.ds(i * _BLK, _BLK)] = probs

        fg = lab != 0
        oh_lab = ((riota == lab) & fg).astype(jnp.float32)
        oh_pred = ((riota == pred) & fg).astype(jnp.float32)
        ct_scr[...] += jax.lax.dot_general(
            oh_pred, oh_lab, (((1,), (1,)), ((), ())),
            preferred_element_type=jnp.float32)
        rt_scr[...] += jax.lax.dot_general(
            probs, oh_lab, (((1,), (1,)), ((), ())),
            preferred_element_type=jnp.float32)

    @pl.when(phase == 1)
    def _pass1():
        @pl.when(i == 0)
        def _solve():
            ct = ct_scr[...]
            m_t = ct + rcpt_ref[...]
            cn = m_t / jnp.sum(m_t, axis=1, keepdims=True)
            cnt_col = jnp.sum(ct, axis=0, keepdims=True)
            r2 = jax.lax.broadcasted_iota(jnp.int32, (_NF, _NF), 0)
            c2 = jax.lax.broadcasted_iota(jnp.int32, (_NF, _NF), 1)
            eye = (r2 == c2).astype(jnp.float32)
            sp_col = jnp.sum(rt_scr[...] * eye, axis=0, keepdims=True)
            lp = sp_col / cnt_col
            rlp = rlp_ref[...]
            rlp_new = jnp.where(cnt_col > 0.0,
                                _MOM * lp + (1.0 - _MOM) * rlp, rlp)
            rp = jax.lax.dot_general(
                rlp_new, cn, (((1,), (0,)), ((), ())),
                preferred_element_type=jnp.float32)
            w2_scr[...] = cn / (rp + _EPS)

        pt = pt_scr[:, pl.ds(i * _BLK, _BLK)]
        out_ref[...] = jax.lax.dot_general(
            pt, w2_scr[...], (((0,), (0,)), ((), ())),
            preferred_element_type=jnp.float32)


def kernel(relation_logits, rel_labels, running_labeling_prob,
           running_column_prob, running_label):
    b, nf = relation_logits.shape
    nb = b // _BLK
    lt = relation_logits.T
    labels3 = rel_labels.reshape(nb, 1, _BLK)
    rlp2 = running_labeling_prob.reshape(1, nf)
    rcpt = running_column_prob.T
    return pl.pallas_call(
        _fused_kernel,
        grid=(2, nb),
        in_specs=[
            pl.BlockSpec((nf, _BLK), lambda p, i: (0, i * (1 - p))),
            pl.BlockSpec((1, 1, _BLK), lambda p, i: (i * (1 - p), 0, 0)),
            pl.BlockSpec((1, nf), lambda p, i: (0, 0)),
            pl.BlockSpec((nf, nf), lambda p, i: (0, 0)),
        ],
        out_specs=pl.BlockSpec((_BLK, nf), lambda p, i: (i * p, 0)),
        out_shape=jax.ShapeDtypeStruct((b, nf), jnp.float32),
        scratch_shapes=[
            pltpu.VMEM((nf, b), jnp.float32),
            pltpu.VMEM((nf, nf), jnp.float32),
            pltpu.VMEM((nf, nf), jnp.float32),
            pltpu.VMEM((nf, nf), jnp.float32),
        ],
    )(lt, labels3, rlp2, rcpt)


# speedup vs baseline: 2.1787x; 1.0538x over previous
import jax
import jax.numpy as jnp
from jax.experimental import pallas as pl
from jax.experimental.pallas import tpu as pltpu

_NF = 51
_BLK = 8192
_MOM = 0.1
_EPS = 1e-5


def _fused_kernel(lt_ref, labels_ref, rlp_ref, rcpt_ref, out_ref,
                  pt_scr, ct_scr, rt_scr, w2_scr):
    phase = pl.program_id(0)
    i = pl.program_id(1)

    @pl.when(phase == 0)
    def _pass0():
        @pl.when(i == 0)
        def _init():
            ct_scr[...] = jnp.zeros_like(ct_scr)
            rt_scr[...] = jnp.zeros_like(rt_scr)

        lt = lt_ref[...]
        lab = labels_ref[0]
        riota = jax.lax.broadcasted_iota(jnp.int32, (_NF, _BLK), 0)

        ml = jnp.where(riota >= 1, lt, -jnp.inf)
        mx = jnp.max(ml, axis=0, keepdims=True)
        pred = jnp.min(jnp.where(ml == mx, riota, _NF + 1), axis=0,
                       keepdims=True)

        mfull = jnp.maximum(mx, lt[0:1, :])
        e = jnp.exp(lt - mfull)
        probs = e / jnp.sum(e, axis=0, keepdims=True)
        pt_scr[:, pl.ds(i * _BLK, _BLK)] = probs

        fg = lab != 0
        oh_lab = ((riota == lab) & fg).astype(jnp.float32)
        oh_pred = ((riota == pred) & fg).astype(jnp.float32)
        ct_scr[...] += jax.lax.dot_general(
            oh_pred, oh_lab, (((1,), (1,)), ((), ())),
            preferred_element_type=jnp.float32)
        rt_scr[...] += jax.lax.dot_general(
            probs, oh_lab, (((1,), (1,)), ((), ())),
            preferred_element_type=jnp.float32)

    @pl.when(phase == 1)
    def _pass1():
        @pl.when(i == 0)
        def _solve():
            ct = ct_scr[...]
            m_t = ct + rcpt_ref[...]
            cn = m_t / jnp.sum(m_t, axis=1, keepdims=True)
            cnt_col = jnp.sum(ct, axis=0, keepdims=True)
            r2 = jax.lax.broadcasted_iota(jnp.int32, (_NF, _NF), 0)
            c2 = jax.lax.broadcasted_iota(jnp.int32, (_NF, _NF), 1)
            eye = (r2 == c2).astype(jnp.float32)
            sp_col = jnp.sum(rt_scr[...] * eye, axis=0, keepdims=True)
            lp = sp_col / cnt_col
            rlp = rlp_ref[...]
            rlp_new = jnp.where(cnt_col > 0.0,
                                _MOM * lp + (1.0 - _MOM) * rlp, rlp)
            rp = jax.lax.dot_general(
                rlp_new, cn, (((1,), (0,)), ((), ())),
                preferred_element_type=jnp.float32)
            w2_scr[...] = cn / (rp + _EPS)

        pt = pt_scr[:, pl.ds(i * _BLK, _BLK)]
        out_ref[...] = jax.lax.dot_general(
            pt, w2_scr[...], (((0,), (0,)), ((), ())),
            preferred_element_type=jnp.float32)


def kernel(relation_logits, rel_labels, running_labeling_prob,
           running_column_prob, running_label):
    b, nf = relation_logits.shape
    nb = b // _BLK
    lt = relation_logits.T
    labels3 = rel_labels.reshape(nb, 1, _BLK)
    rlp2 = running_labeling_prob.reshape(1, nf)
    rcpt = running_column_prob.T
    return pl.pallas_call(
        _fused_kernel,
        grid=(2, nb),
        in_specs=[
            pl.BlockSpec((nf, _BLK), lambda p, i: (0, i * (1 - p))),
            pl.BlockSpec((1, 1, _BLK), lambda p, i: (i * (1 - p), 0, 0)),
            pl.BlockSpec((1, nf), lambda p, i: (0, 0)),
            pl.BlockSpec((nf, nf), lambda p, i: (0, 0)),
        ],
        out_specs=pl.BlockSpec((_BLK, nf), lambda p, i: (i * p, 0)),
        out_shape=jax.ShapeDtypeStruct((b, nf), jnp.float32),
        scratch_shapes=[
            pltpu.VMEM((nf, b), jnp.float32),
            pltpu.VMEM((nf, nf), jnp.float32),
            pltpu.VMEM((nf, nf), jnp.float32),
            pltpu.VMEM((nf, nf), jnp.float32),
        ],
    )(lt, labels3, rlp2, rcpt)
